# baseline (device time: 47378 ns/iter reference)
import jax
import jax.numpy as jnp
from jax import lax
from jax.experimental import pallas as pl
from jax.experimental.pallas import tpu as pltpu

N_Z = 4


def kernel(partial, resid, gamma):
    m, d = resid.shape
    x_shard = partial.reshape(m, d)
    gamma2d = gamma.reshape(1, d)

    def body(x_ref, resid_ref, gamma_ref, out_ref, comm_ref, send_sems, recv_sems):
        my_x = lax.axis_index("x")
        my_y = lax.axis_index("y")
        my_z = lax.axis_index("z")
        left = (my_z - 1) % N_Z
        right = (my_z + 1) % N_Z

        barrier_sem = pltpu.get_barrier_semaphore()
        for nbr in (left, right):
            pl.semaphore_signal(
                barrier_sem, inc=1,
                device_id=(my_x, my_y, nbr),
                device_id_type=pl.DeviceIdType.MESH,
            )
        pl.semaphore_wait(barrier_sem, 2)

        out_ref[:, :] = x_ref[:, :]
        comm_ref[0, :, :] = x_ref[:, :]

        for h in range(N_Z - 1):
            send_slot = h % 2
            recv_slot = (h + 1) % 2
            rdma = pltpu.make_async_remote_copy(
                src_ref=comm_ref.at[send_slot],
                dst_ref=comm_ref.at[recv_slot],
                send_sem=send_sems.at[send_slot],
                recv_sem=recv_sems.at[recv_slot],
                device_id=(my_x, my_y, right),
                device_id_type=pl.DeviceIdType.MESH,
            )
            rdma.start()
            rdma.wait()
            out_ref[:, :] += comm_ref[recv_slot, :, :]

        y = out_ref[:, :] + resid_ref[:, :]
        rms = jnp.sqrt(jnp.mean(y * y, axis=-1, keepdims=True) + 1e-6)
        out_ref[:, :] = y / rms * gamma_ref[:, :]

    return pl.pallas_call(
        body,
        out_shape=jax.ShapeDtypeStruct((m, d), jnp.float32),
        in_specs=[
            pl.BlockSpec(memory_space=pltpu.VMEM),
            pl.BlockSpec(memory_space=pltpu.VMEM),
            pl.BlockSpec(memory_space=pltpu.VMEM),
        ],
        out_specs=pl.BlockSpec(memory_space=pltpu.VMEM),
        scratch_shapes=[
            pltpu.VMEM((2, m, d), jnp.float32),
            pltpu.SemaphoreType.DMA((2,)),
            pltpu.SemaphoreType.DMA((2,)),
        ],
        compiler_params=pltpu.CompilerParams(collective_id=0),
    )(x_shard, resid, gamma2d)


# device time: 28599 ns/iter; 1.6566x vs baseline; 1.6566x over previous
import jax
import jax.numpy as jnp
from jax import lax
from jax.experimental import pallas as pl
from jax.experimental.pallas import tpu as pltpu

N_Z = 4
N_PLANE = 8
N_Y = 4
ROWS = 16


def kernel(partial, resid, gamma):
    m, d = resid.shape
    x_shard = partial.reshape(m, d)
    gamma2d = gamma.reshape(1, d)

    def body(x_ref, resid_ref, gamma_ref, out_ref, zbuf,
             zr_send, zr_recv, g1_send, g1_recv, g2_send, g2_recv):
        my_x = lax.axis_index("x")
        my_y = lax.axis_index("y")
        my_z = lax.axis_index("z")
        c = my_x * N_Y + my_y
        p_off = ROWS * (c * N_Z + my_z)
        blk_off = ROWS * N_Z * c

        def z_peer(r):
            return (my_x, my_y, lax.rem(my_z + r, N_Z))

        def plane_peer(rho):
            cc = lax.rem(c + rho, N_PLANE)
            return (cc // N_Y, lax.rem(cc, N_Y), my_z)

        barrier_sem = pltpu.get_barrier_semaphore()
        for r in range(1, N_Z):
            pl.semaphore_signal(barrier_sem, inc=1, device_id=z_peer(r),
                                device_id_type=pl.DeviceIdType.MESH)
        for rho in range(1, N_PLANE):
            pl.semaphore_signal(barrier_sem, inc=1, device_id=plane_peer(rho),
                                device_id_type=pl.DeviceIdType.MESH)
        pl.semaphore_wait(barrier_sem, (N_Z - 1) + (N_PLANE - 1))

        z_sends = []
        for r in range(1, N_Z):
            peer_part = ROWS * (c * N_Z + lax.rem(my_z + r, N_Z))
            rdma = pltpu.make_async_remote_copy(
                src_ref=x_ref.at[pl.ds(peer_part, ROWS), :],
                dst_ref=zbuf.at[N_Z - r],
                send_sem=zr_send.at[r],
                recv_sem=zr_recv.at[N_Z - r],
                device_id=z_peer(r),
                device_id_type=pl.DeviceIdType.MESH,
            )
            rdma.start()
            z_sends.append(rdma)

        for s in range(1, N_Z):
            recv = pltpu.make_async_remote_copy(
                src_ref=x_ref.at[pl.ds(0, ROWS), :],
                dst_ref=zbuf.at[s],
                send_sem=zr_send.at[s],
                recv_sem=zr_recv.at[s],
                device_id=z_peer(1),
                device_id_type=pl.DeviceIdType.MESH,
            )
            recv.wait_recv()

        red = x_ref[pl.ds(p_off, ROWS), :] + zbuf[1] + zbuf[2] + zbuf[3]
        y = red + resid_ref[pl.ds(p_off, ROWS), :]
        rms = jnp.sqrt(jnp.mean(y * y, axis=-1, keepdims=True) + 1e-6)
        out_ref[pl.ds(p_off, ROWS), :] = y / rms * gamma_ref[:, :]

        g1_sends = []
        for r in range(1, N_Z):
            rdma = pltpu.make_async_remote_copy(
                src_ref=out_ref.at[pl.ds(p_off, ROWS), :],
                dst_ref=out_ref.at[pl.ds(p_off, ROWS), :],
                send_sem=g1_send.at[r],
                recv_sem=g1_recv.at[N_Z - r],
                device_id=z_peer(r),
                device_id_type=pl.DeviceIdType.MESH,
            )
            rdma.start()
            g1_sends.append(rdma)

        for s in range(1, N_Z):
            src_part = ROWS * (c * N_Z + lax.rem(my_z + N_Z - s, N_Z))
            recv = pltpu.make_async_remote_copy(
                src_ref=out_ref.at[pl.ds(0, ROWS), :],
                dst_ref=out_ref.at[pl.ds(src_part, ROWS), :],
                send_sem=g1_send.at[s],
                recv_sem=g1_recv.at[s],
                device_id=z_peer(1),
                device_id_type=pl.DeviceIdType.MESH,
            )
            recv.wait_recv()

        g2_sends = []
        for rho in range(1, N_PLANE):
            rdma = pltpu.make_async_remote_copy(
                src_ref=out_ref.at[pl.ds(blk_off, ROWS * N_Z), :],
                dst_ref=out_ref.at[pl.ds(blk_off, ROWS * N_Z), :],
                send_sem=g2_send.at[rho],
                recv_sem=g2_recv.at[N_PLANE - rho],
                device_id=plane_peer(rho),
                device_id_type=pl.DeviceIdType.MESH,
            )
            rdma.start()
            g2_sends.append(rdma)

        for s in range(1, N_PLANE):
            src_blk = ROWS * N_Z * lax.rem(c + N_PLANE - s, N_PLANE)
            recv = pltpu.make_async_remote_copy(
                src_ref=out_ref.at[pl.ds(0, ROWS * N_Z), :],
                dst_ref=out_ref.at[pl.ds(src_blk, ROWS * N_Z), :],
                send_sem=g2_send.at[s],
                recv_sem=g2_recv.at[s],
                device_id=plane_peer(1),
                device_id_type=pl.DeviceIdType.MESH,
            )
            recv.wait_recv()

        for rdma in z_sends + g1_sends + g2_sends:
            rdma.wait_send()

    return pl.pallas_call(
        body,
        out_shape=jax.ShapeDtypeStruct((m, d), jnp.float32),
        in_specs=[
            pl.BlockSpec(memory_space=pltpu.VMEM),
            pl.BlockSpec(memory_space=pltpu.VMEM),
            pl.BlockSpec(memory_space=pltpu.VMEM),
        ],
        out_specs=pl.BlockSpec(memory_space=pltpu.VMEM),
        scratch_shapes=[
            pltpu.VMEM((N_Z, ROWS, d), jnp.float32),
            pltpu.SemaphoreType.DMA((N_Z,)),
            pltpu.SemaphoreType.DMA((N_Z,)),
            pltpu.SemaphoreType.DMA((N_Z,)),
            pltpu.SemaphoreType.DMA((N_Z,)),
            pltpu.SemaphoreType.DMA((N_PLANE,)),
            pltpu.SemaphoreType.DMA((N_PLANE,)),
        ],
        compiler_params=pltpu.CompilerParams(collective_id=0),
    )(x_shard, resid, gamma2d)
